# baseline (speedup 1.0000x reference)
import functools

import jax
import jax.numpy as jnp
from jax import lax
from jax.experimental import pallas as pl
from jax.experimental.pallas import tpu as pltpu, tpu_sc as plsc

FIELDS = 26
VOCAB = 100000
EMB = 16
B = 4096
EPS = 1e-5

NC = 2
NS = 16
NW = NC * NS
BPW = B // NW
PAIRS_W = BPW * FIELDS
NCHUNKS = PAIRS_W // 128

BLK = 512
D = FIELDS * EMB


def _gather_body(fm2_hbm, fm1_hbm, idx_hbm, rows_hbm, fm1g_hbm,
                 idx_v, rows_v, fm1g_v, sem_a, sem_b):
    wid = lax.axis_index("s") * NC + lax.axis_index("c")

    pltpu.sync_copy(idx_hbm.at[pl.ds(wid * NCHUNKS, NCHUNKS)], idx_v)
    g2s = [
        pltpu.async_copy(fm2_hbm.at[idx_v.at[j]],
                         rows_v.at[pl.ds(j * 128, 128)], sem_a)
        for j in range(NCHUNKS)
    ]
    g1s = [
        pltpu.async_copy(fm1_hbm.at[idx_v.at[j]],
                         fm1g_v.at[pl.ds(j * 128, 128)], sem_b)
        for j in range(NCHUNKS)
    ]
    for h in g2s:
        h.wait()
    for h in g1s:
        h.wait()
    pltpu.sync_copy(rows_v, rows_hbm.at[pl.ds(wid * PAIRS_W, PAIRS_W)])
    pltpu.sync_copy(fm1g_v, fm1g_hbm.at[pl.ds(wid * PAIRS_W, PAIRS_W)])


def _make_sc_gather():
    mesh = plsc.VectorSubcoreMesh(core_axis_name="c", subcore_axis_name="s")
    return functools.partial(
        pl.kernel,
        mesh=mesh,
        compiler_params=pltpu.CompilerParams(use_tc_tiling_on_sc=False),
        out_type=(
            jax.ShapeDtypeStruct((B * FIELDS, EMB), jnp.float32),
            jax.ShapeDtypeStruct((B * FIELDS,), jnp.float32),
        ),
        scratch_types=[
            pltpu.VMEM((NCHUNKS, 128), jnp.int32),
            pltpu.VMEM((PAIRS_W, EMB), jnp.float32),
            pltpu.VMEM((PAIRS_W,), jnp.float32),
            pltpu.SemaphoreType.DMA,
            pltpu.SemaphoreType.DMA,
        ],
    )(_gather_body)


def _dense_body(rows_ref, fm1g_ref, xv_ref, e_ref, m_ref, c_ref, o_ref):
    xve = jnp.dot(xv_ref[...], e_ref[...],
                  preferred_element_type=jnp.float32)
    xe = rows_ref[...] * xve
    sl = jnp.dot(xe, m_ref[...],
                 preferred_element_type=jnp.float32)
    s = sl[:, :EMB]
    deep = sl[:, EMB:]
    q = jnp.sum(xe * xe, axis=1, keepdims=True)
    s2 = jnp.sum(s * s, axis=1, keepdims=True)
    first = jnp.sum(fm1g_ref[...] * xv_ref[...], axis=1, keepdims=True)
    o_ref[...] = first + 0.5 * (s2 - q) + deep + c_ref[...]


def _dense_stage(rows, fm1g, xv, emat, mmat, cmat):
    grid = (B // BLK,)
    return pl.pallas_call(
        _dense_body,
        grid=grid,
        in_specs=[
            pl.BlockSpec((BLK, D), lambda i: (i, 0)),
            pl.BlockSpec((BLK, FIELDS), lambda i: (i, 0)),
            pl.BlockSpec((BLK, FIELDS), lambda i: (i, 0)),
            pl.BlockSpec((FIELDS, D), lambda i: (0, 0)),
            pl.BlockSpec((D, EMB + 1), lambda i: (0, 0)),
            pl.BlockSpec((1, 1), lambda i: (0, 0)),
        ],
        out_specs=pl.BlockSpec((BLK, 1), lambda i: (i, 0)),
        out_shape=jax.ShapeDtypeStruct((B, 1), jnp.float32),
    )(rows, fm1g, xv, emat, mmat, cmat)


_sc_gather_cache = None


def _get_sc_gather():
    global _sc_gather_cache
    if _sc_gather_cache is None:
        _sc_gather_cache = _make_sc_gather()
    return _sc_gather_cache


def kernel(Xi, Xv, fm1, fm2, W1, b1, g1, be1, W2, b2, g2, be2, bias):
    s = 1.0 / jnp.sqrt(1.0 + EPS)
    u = (s * g2) @ W2
    a1 = s * g1 * u
    w_eff = W1.T @ a1
    const = b1 @ a1 + be1 @ u + b2 @ (s * g2) + be2.sum() + bias[0]

    emat = jnp.repeat(jnp.eye(FIELDS, dtype=jnp.float32), EMB, axis=1)
    pmat = jnp.tile(jnp.eye(EMB, dtype=jnp.float32), (FIELDS, 1))
    mmat = jnp.concatenate(
        [pmat, w_eff.astype(jnp.float32)[:, None]], axis=1)
    cmat = jnp.broadcast_to(const, (1, 1)).astype(jnp.float32)

    idx = Xi[:, :, 0].astype(jnp.int32)
    flat = idx + (jnp.arange(FIELDS, dtype=jnp.int32) * VOCAB)[None, :]
    idx_w = flat.reshape(NW * NCHUNKS, 128)

    fm2_flat = fm2.reshape(FIELDS * VOCAB, EMB)
    fm1_flat = fm1.reshape(FIELDS * VOCAB)

    rows_g, fm1_g = _get_sc_gather()(fm2_flat, fm1_flat, idx_w)
    rows = rows_g.reshape(B, D)
    fm1g = fm1_g.reshape(B, FIELDS)

    out = _dense_stage(rows, fm1g, Xv.astype(jnp.float32), emat, mmat, cmat)
    return out.reshape(B)
